# baseline (speedup 1.0000x reference)
import functools

import jax
import jax.numpy as jnp
from jax.experimental import pallas as pl

N_TOKENS = 8192
D_MODEL = 768
N_EXPERTS = 8
BLOCK_T = 1024


def _moe_body(x_ref, gw_ref, ewf_ref, eb_ref, o_ref):
    x = x_ref[...]
    logits = jnp.dot(x, gw_ref[...], preferred_element_type=jnp.float32)
    w = jax.nn.softmax(logits, axis=-1)
    xs = jnp.concatenate(
        [(x * w[:, e : e + 1]).astype(jnp.bfloat16) for e in range(N_EXPERTS)],
        axis=1,
    )
    acc = jnp.dot(xs, ewf_ref[...], preferred_element_type=jnp.float32)
    acc = acc + jnp.dot(w, eb_ref[...], preferred_element_type=jnp.float32)
    o_ref[...] = acc.astype(o_ref.dtype)


@functools.partial(jax.jit, static_argnames=("interpret",))
def kernel(inputs, gate_w, expert_w, expert_b, interpret=False):
    n_tokens, d_model = inputs.shape
    n_experts = expert_w.shape[0]
    ew_flat = expert_w.reshape(n_experts * d_model, d_model)
    grid = (n_tokens // BLOCK_T,)
    return pl.pallas_call(
        _moe_body,
        grid=grid,
        in_specs=[
            pl.BlockSpec((BLOCK_T, d_model), lambda i: (i, 0)),
            pl.BlockSpec((d_model, n_experts), lambda i: (0, 0)),
            pl.BlockSpec((n_experts * d_model, d_model), lambda i: (0, 0)),
            pl.BlockSpec((n_experts, d_model), lambda i: (0, 0)),
        ],
        out_specs=pl.BlockSpec((BLOCK_T, d_model), lambda i: (i, 0)),
        out_shape=jax.ShapeDtypeStruct((n_tokens, d_model), inputs.dtype),
        interpret=interpret,
    )(inputs, gate_w, ew_flat, expert_b)


# speedup vs baseline: 1.0505x; 1.0505x over previous
import functools

import jax
import jax.numpy as jnp
from jax.experimental import pallas as pl

N_TOKENS = 8192
D_MODEL = 768
N_EXPERTS = 8
BLOCK_T = 1024


def _moe_body(x_ref, gw_ref, ew_ref, eb_ref, o_ref):
    x = x_ref[...]
    logits = jnp.dot(x, gw_ref[...], preferred_element_type=jnp.float32)
    w = jax.nn.softmax(logits, axis=-1)
    acc = jnp.dot(w, eb_ref[...], preferred_element_type=jnp.float32)
    for e in range(N_EXPERTS):
        xe = x * w[:, e : e + 1]
        acc = acc + jnp.dot(xe, ew_ref[e], preferred_element_type=jnp.float32)
    o_ref[...] = acc.astype(o_ref.dtype)


@functools.partial(jax.jit, static_argnames=("interpret",))
def kernel(inputs, gate_w, expert_w, expert_b, interpret=False):
    n_tokens, d_model = inputs.shape
    n_experts = expert_w.shape[0]
    grid = (n_tokens // BLOCK_T,)
    return pl.pallas_call(
        _moe_body,
        grid=grid,
        in_specs=[
            pl.BlockSpec((BLOCK_T, d_model), lambda i: (i, 0)),
            pl.BlockSpec((d_model, n_experts), lambda i: (0, 0)),
            pl.BlockSpec((n_experts, d_model, d_model), lambda i: (0, 0, 0)),
            pl.BlockSpec((n_experts, d_model), lambda i: (0, 0)),
        ],
        out_specs=pl.BlockSpec((BLOCK_T, d_model), lambda i: (i, 0)),
        out_shape=jax.ShapeDtypeStruct((n_tokens, d_model), inputs.dtype),
        interpret=interpret,
    )(inputs, gate_w, expert_w, expert_b)


# speedup vs baseline: 1.1223x; 1.0684x over previous
import functools

import jax
import jax.numpy as jnp
from jax.experimental import pallas as pl

N_TOKENS = 8192
D_MODEL = 768
N_EXPERTS = 8
BLOCK_T = 1024


def _moe_body(x_ref, gw_ref, ew_ref, eb_ref, o_ref):
    x = x_ref[...]
    logits = jnp.dot(x, gw_ref[...], preferred_element_type=jnp.float32)
    w = jax.nn.softmax(logits, axis=-1)
    acc = jnp.dot(w, eb_ref[...], preferred_element_type=jnp.float32)
    for e in range(N_EXPERTS):
        y = jnp.dot(x, ew_ref[e], preferred_element_type=jnp.float32)
        acc = acc + w[:, e : e + 1] * y
    o_ref[...] = acc.astype(o_ref.dtype)


@functools.partial(jax.jit, static_argnames=("interpret",))
def kernel(inputs, gate_w, expert_w, expert_b, interpret=False):
    n_tokens, d_model = inputs.shape
    n_experts = expert_w.shape[0]
    grid = (n_tokens // BLOCK_T,)
    return pl.pallas_call(
        _moe_body,
        grid=grid,
        in_specs=[
            pl.BlockSpec((BLOCK_T, d_model), lambda i: (i, 0)),
            pl.BlockSpec((d_model, n_experts), lambda i: (0, 0)),
            pl.BlockSpec((n_experts, d_model, d_model), lambda i: (0, 0, 0)),
            pl.BlockSpec((n_experts, d_model), lambda i: (0, 0)),
        ],
        out_specs=pl.BlockSpec((BLOCK_T, d_model), lambda i: (i, 0)),
        out_shape=jax.ShapeDtypeStruct((n_tokens, d_model), inputs.dtype),
        interpret=interpret,
    )(inputs, gate_w, expert_w, expert_b)
